# baseline (speedup 1.0000x reference)
import math

import jax
import jax.numpy as jnp
from jax.experimental import pallas as pl
from jax.experimental.pallas import tpu as pltpu

INV_SQRT2 = 1.0 / math.sqrt(2.0)
NEG_SLOPE = 0.2

H = 32
W = 32
HW = H * W
PLANES = 32
TAPS = 9
IMGS_PER_STEP = 4
M1 = TAPS * PLANES
C1 = 256


def _lrelu(v):
    return jnp.where(v >= 0.0, v, NEG_SLOPE * v)


def _combine_taps(parts):
    q = jax.lax.broadcasted_iota(jnp.int32, (PLANES, HW), 1)
    hh = q // W
    ww = q % W
    acc = None
    for t in range(TAPS):
        dy = t // 3 - 1
        dx = t % 3 - 1
        z = parts[t * PLANES:(t + 1) * PLANES, :]
        off = dy * W + dx
        if off != 0:
            z = jnp.roll(z, -off, axis=1)
        cond = None
        for c in ((hh >= 1) if dy == -1 else None,
                  (hh <= H - 2) if dy == 1 else None,
                  (ww >= 1) if dx == -1 else None,
                  (ww <= W - 2) if dx == 1 else None):
            if c is not None:
                cond = c if cond is None else (cond & c)
        if cond is not None:
            z = jnp.where(cond, z, 0.0)
        acc = z if acc is None else acc + z
    return acc


def _block_kernel(x_ref, wt_ref, wp_ref, bias_ref, coords_ref, o_ref):
    w1m = wt_ref[:, :C1]
    w1c = wt_ref[:, C1:C1 + 2]
    w2m = wt_ref[:, C1 + 2:C1 + 2 + PLANES]
    w2c = wt_ref[:, C1 + 2 + PLANES:]
    wp = wp_ref[...]
    b1 = bias_ref[:, 0:1]
    b2 = bias_ref[:, 1:2]
    bp = bias_ref[:, 2:3]
    coords = coords_ref[...]

    for img in range(IMGS_PER_STEP):
        x = x_ref[img]

        a = jnp.dot(w1m, x, preferred_element_type=jnp.float32)
        a = a + jnp.dot(w1c, coords, preferred_element_type=jnp.float32)
        y1 = _lrelu(_combine_taps(a) + b1)

        ident = jnp.dot(wp, x, preferred_element_type=jnp.float32) + bp

        b = jnp.dot(w2m, y1.astype(jnp.bfloat16),
                    preferred_element_type=jnp.float32)
        b = b + jnp.dot(w2c, coords, preferred_element_type=jnp.float32)
        y2 = _lrelu(_combine_taps(b) + b2)

        o_ref[img] = ((y2 + ident) * INV_SQRT2).reshape(PLANES, H, W)


def _tap_major(w):
    cout, cin = w.shape[0], w.shape[1]
    return w.transpose(2, 3, 0, 1).reshape(TAPS * cout, cin)


def kernel(w1, b1, w2, b2, wproj, bproj, x):
    B, Cin = x.shape[0], x.shape[1]
    x3 = x.reshape(B, Cin, HW).astype(jnp.bfloat16)

    wcat = jnp.concatenate(
        [w1.astype(jnp.float32), w2.astype(jnp.float32)], axis=1)
    wt = _tap_major(wcat).astype(jnp.bfloat16)
    wp = wproj.astype(jnp.bfloat16).reshape(PLANES, Cin)

    bias = jnp.stack([b1, b2, bproj], axis=1).astype(jnp.float32)

    span = jnp.arange(H, dtype=jnp.float32) / (H - 1) * 2.0 - 1.0
    xx = jnp.broadcast_to(span[:, None], (H, W)).reshape(1, HW)
    yy = jnp.broadcast_to(span[None, :], (H, W)).reshape(1, HW)
    coords = jnp.concatenate([xx, yy], axis=0).astype(jnp.bfloat16)

    out = pl.pallas_call(
        _block_kernel,
        grid=(B // IMGS_PER_STEP,),
        out_shape=jax.ShapeDtypeStruct((B, PLANES, H, W), jnp.float32),
        in_specs=[
            pl.BlockSpec((IMGS_PER_STEP, Cin, HW), lambda i: (i, 0, 0)),
            pl.BlockSpec(wt.shape, lambda i: (0, 0)),
            pl.BlockSpec(wp.shape, lambda i: (0, 0)),
            pl.BlockSpec(bias.shape, lambda i: (0, 0)),
            pl.BlockSpec(coords.shape, lambda i: (0, 0)),
        ],
        out_specs=pl.BlockSpec((IMGS_PER_STEP, PLANES, H, W),
                               lambda i: (i, 0, 0, 0)),
        compiler_params=pltpu.CompilerParams(
            dimension_semantics=("parallel",)),
    )(x3, wt, wp, bias, coords)

    return out


# speedup vs baseline: 1.1510x; 1.1510x over previous
import math

import jax
import jax.numpy as jnp
from jax.experimental import pallas as pl
from jax.experimental.pallas import tpu as pltpu

INV_SQRT2 = 1.0 / math.sqrt(2.0)
NEG_SLOPE = 0.2

H = 32
W = 32
HW = H * W
PLANES = 32
TAPS = 9
IMGS_PER_STEP = 4
M1 = TAPS * PLANES
C1 = 256


def _lrelu(v):
    return jnp.where(v >= 0.0, v, NEG_SLOPE * v)


def _combine_taps(parts):
    q = jax.lax.broadcasted_iota(jnp.int32, (PLANES, HW), 1)
    hh = q // W
    ww = q % W
    acc = None
    for t in range(TAPS):
        dy = t // 3 - 1
        dx = t % 3 - 1
        z = parts[t * PLANES:(t + 1) * PLANES, :]
        off = dy * W + dx
        if off != 0:
            z = jnp.roll(z, -off, axis=1)
        cond = None
        for c in ((hh >= 1) if dy == -1 else None,
                  (hh <= H - 2) if dy == 1 else None,
                  (ww >= 1) if dx == -1 else None,
                  (ww <= W - 2) if dx == 1 else None):
            if c is not None:
                cond = c if cond is None else (cond & c)
        if cond is not None:
            z = jnp.where(cond, z, 0.0)
        acc = z if acc is None else acc + z
    return acc


def _block_kernel(x_ref, wt_ref, wp_ref, bias_ref, coords_ref, o_ref):
    w1m = wt_ref[:, :C1]
    w1c = wt_ref[:, C1:C1 + 2]
    w2m = wt_ref[:, C1 + 2:C1 + 2 + PLANES]
    w2c = wt_ref[:, C1 + 2 + PLANES:]
    wp = wp_ref[...]
    b1 = bias_ref[:, 0:1]
    b2 = bias_ref[:, 1:2]
    bp = bias_ref[:, 2:3]
    coords = coords_ref[...]

    c1 = jnp.dot(w1c, coords, preferred_element_type=jnp.float32)
    c2 = jnp.dot(w2c, coords, preferred_element_type=jnp.float32)

    y1s = []
    idents = []
    for img in range(IMGS_PER_STEP):
        x = x_ref[img].astype(jnp.bfloat16)
        a = jnp.dot(w1m, x, preferred_element_type=jnp.float32) + c1
        y1s.append(_lrelu(_combine_taps(a) + b1).astype(jnp.bfloat16))
        idents.append(jnp.dot(wp, x, preferred_element_type=jnp.float32) + bp)

    y1cat = jnp.concatenate(y1s, axis=1)
    bcat = jnp.dot(w2m, y1cat, preferred_element_type=jnp.float32)

    for img in range(IMGS_PER_STEP):
        b = bcat[:, img * HW:(img + 1) * HW] + c2
        y2 = _lrelu(_combine_taps(b) + b2)
        o_ref[img] = ((y2 + idents[img]) * INV_SQRT2).reshape(PLANES, H, W)


def _tap_major(w):
    cout, cin = w.shape[0], w.shape[1]
    return w.transpose(2, 3, 0, 1).reshape(TAPS * cout, cin)


def kernel(w1, b1, w2, b2, wproj, bproj, x):
    B, Cin = x.shape[0], x.shape[1]
    x3 = x.astype(jnp.float32).reshape(B, Cin, HW)

    wcat = jnp.concatenate(
        [w1.astype(jnp.bfloat16), w2.astype(jnp.bfloat16)], axis=1)
    wt = _tap_major(wcat)
    wp = wproj.astype(jnp.bfloat16).reshape(PLANES, Cin)

    bias = jnp.stack([b1, b2, bproj], axis=1).astype(jnp.float32)

    span = jnp.arange(H, dtype=jnp.float32) / (H - 1) * 2.0 - 1.0
    xx = jnp.broadcast_to(span[:, None], (H, W)).reshape(1, HW)
    yy = jnp.broadcast_to(span[None, :], (H, W)).reshape(1, HW)
    coords = jnp.concatenate([xx, yy], axis=0).astype(jnp.bfloat16)

    out = pl.pallas_call(
        _block_kernel,
        grid=(B // IMGS_PER_STEP,),
        out_shape=jax.ShapeDtypeStruct((B, PLANES, H, W), jnp.float32),
        in_specs=[
            pl.BlockSpec((IMGS_PER_STEP, Cin, HW), lambda i: (i, 0, 0)),
            pl.BlockSpec(wt.shape, lambda i: (0, 0)),
            pl.BlockSpec(wp.shape, lambda i: (0, 0)),
            pl.BlockSpec(bias.shape, lambda i: (0, 0)),
            pl.BlockSpec(coords.shape, lambda i: (0, 0)),
        ],
        out_specs=pl.BlockSpec((IMGS_PER_STEP, PLANES, H, W),
                               lambda i: (i, 0, 0, 0)),
        compiler_params=pltpu.CompilerParams(
            dimension_semantics=("parallel",)),
    )(x3, wt, wp, bias, coords)

    return out


# speedup vs baseline: 1.1729x; 1.0190x over previous
import math

import jax
import jax.numpy as jnp
from jax.experimental import pallas as pl
from jax.experimental.pallas import tpu as pltpu

INV_SQRT2 = 1.0 / math.sqrt(2.0)
NEG_SLOPE = 0.2

H = 32
W = 32
HW = H * W
PLANES = 32
TAPS = 9
IMGS_PER_STEP = 8
M1 = TAPS * PLANES
MBIG = M1 + PLANES
C1 = 256


def _lrelu(v):
    return jnp.where(v >= 0.0, v, NEG_SLOPE * v)


def _combine_taps(parts):
    q = jax.lax.broadcasted_iota(jnp.int32, (PLANES, HW), 1)
    hh = q // W
    ww = q % W
    acc = None
    for t in range(TAPS):
        dy = t // 3 - 1
        dx = t % 3 - 1
        z = parts[t * PLANES:(t + 1) * PLANES, :]
        off = dy * W + dx
        if off != 0:
            z = jnp.roll(z, -off, axis=1)
        cond = None
        for c in ((hh >= 1) if dy == -1 else None,
                  (hh <= H - 2) if dy == 1 else None,
                  (ww >= 1) if dx == -1 else None,
                  (ww <= W - 2) if dx == 1 else None):
            if c is not None:
                cond = c if cond is None else (cond & c)
        if cond is not None:
            z = jnp.where(cond, z, 0.0)
        acc = z if acc is None else acc + z
    return acc


def _block_kernel(x_ref, wt_ref, bias_ref, coords_ref, o_ref):
    wt = wt_ref[...].astype(jnp.bfloat16)
    w1m = wt[:, :C1]
    w1c = wt[:M1, C1:C1 + 2]
    w2m = wt[:M1, C1 + 2:C1 + 2 + PLANES]
    w2c = wt[:M1, C1 + 2 + PLANES:]
    b1 = bias_ref[:, 0:1]
    b2 = bias_ref[:, 1:2]
    bp = bias_ref[:, 2:3]
    coords = coords_ref[...]

    c1 = jnp.dot(w1c, coords, preferred_element_type=jnp.float32)
    c2 = jnp.dot(w2c, coords, preferred_element_type=jnp.float32)

    y1s = []
    idents = []
    for img in range(IMGS_PER_STEP):
        x = x_ref[img].astype(jnp.bfloat16)
        a = jnp.dot(w1m, x, preferred_element_type=jnp.float32)
        y1s.append(_lrelu(_combine_taps(a[:M1] + c1) + b1).astype(jnp.bfloat16))
        idents.append(a[M1:MBIG] + bp)

    y1cat = jnp.concatenate(y1s, axis=1)
    bcat = jnp.dot(w2m, y1cat, preferred_element_type=jnp.float32)

    for img in range(IMGS_PER_STEP):
        b = bcat[:, img * HW:(img + 1) * HW] + c2
        y2 = _lrelu(_combine_taps(b) + b2)
        o_ref[img] = ((y2 + idents[img]) * INV_SQRT2).reshape(PLANES, H, W)


def _tap_major(w):
    cout, cin = w.shape[0], w.shape[1]
    return w.transpose(2, 3, 0, 1).reshape(TAPS * cout, cin)


def kernel(w1, b1, w2, b2, wproj, bproj, x):
    B, Cin = x.shape[0], x.shape[1]
    x3 = x.astype(jnp.float32).reshape(B, Cin, HW)

    wcat = jnp.concatenate(
        [w1.astype(jnp.float32), w2.astype(jnp.float32)], axis=1)
    wt = jnp.concatenate(
        [_tap_major(wcat),
         jnp.pad(wproj.astype(jnp.float32).reshape(PLANES, Cin),
                 ((0, 0), (0, 36)))], axis=0)

    bias = jnp.stack([b1, b2, bproj], axis=1).astype(jnp.float32)

    span = jnp.arange(H, dtype=jnp.float32) / (H - 1) * 2.0 - 1.0
    xx = jnp.broadcast_to(span[:, None], (H, W)).reshape(1, HW)
    yy = jnp.broadcast_to(span[None, :], (H, W)).reshape(1, HW)
    coords = jnp.concatenate([xx, yy], axis=0).astype(jnp.bfloat16)

    out = pl.pallas_call(
        _block_kernel,
        grid=(B // IMGS_PER_STEP,),
        out_shape=jax.ShapeDtypeStruct((B, PLANES, H, W), jnp.float32),
        in_specs=[
            pl.BlockSpec((IMGS_PER_STEP, Cin, HW), lambda i: (i, 0, 0)),
            pl.BlockSpec(wt.shape, lambda i: (0, 0)),
            pl.BlockSpec(bias.shape, lambda i: (0, 0)),
            pl.BlockSpec(coords.shape, lambda i: (0, 0)),
        ],
        out_specs=pl.BlockSpec((IMGS_PER_STEP, PLANES, H, W),
                               lambda i: (i, 0, 0, 0)),
        compiler_params=pltpu.CompilerParams(
            dimension_semantics=("parallel",)),
    )(x3, wt, bias, coords)

    return out


# speedup vs baseline: 1.1925x; 1.0167x over previous
import math

import jax
import jax.numpy as jnp
from jax.experimental import pallas as pl
from jax.experimental.pallas import tpu as pltpu

INV_SQRT2 = 1.0 / math.sqrt(2.0)
NEG_SLOPE = 0.2

H = 32
W = 32
HW = H * W
PLANES = 32
TAPS = 9
IMGS_PER_STEP = 4
M1 = TAPS * PLANES
MBIG = M1 + PLANES
C1 = 256


def _lrelu(v):
    return jnp.where(v >= 0.0, v, NEG_SLOPE * v)


def _combine_taps(parts):
    q = jax.lax.broadcasted_iota(jnp.int32, (PLANES, HW), 1)
    hh = q // W
    ww = q % W
    acc = None
    for t in range(TAPS):
        dy = t // 3 - 1
        dx = t % 3 - 1
        z = parts[t * PLANES:(t + 1) * PLANES, :]
        off = dy * W + dx
        if off != 0:
            z = jnp.roll(z, -off, axis=1)
        cond = None
        for c in ((hh >= 1) if dy == -1 else None,
                  (hh <= H - 2) if dy == 1 else None,
                  (ww >= 1) if dx == -1 else None,
                  (ww <= W - 2) if dx == 1 else None):
            if c is not None:
                cond = c if cond is None else (cond & c)
        if cond is not None:
            z = jnp.where(cond, z, 0.0)
        acc = z if acc is None else acc + z
    return acc


def _block_kernel(x_ref, wt_ref, bias_ref, coords_ref, o_ref):
    wt = wt_ref[...].astype(jnp.bfloat16)
    w1m = wt[:, :C1]
    w1c = wt[:M1, C1:C1 + 2]
    w2m = wt[:M1, C1 + 2:C1 + 2 + PLANES]
    w2c = wt[:M1, C1 + 2 + PLANES:]
    b1 = bias_ref[:, 0:1]
    b2 = bias_ref[:, 1:2]
    bp = bias_ref[:, 2:3]
    coords = coords_ref[...]

    c1 = jnp.dot(w1c, coords, preferred_element_type=jnp.float32)
    c2 = jnp.dot(w2c, coords, preferred_element_type=jnp.float32)

    y1s = []
    idents = []
    for img in range(IMGS_PER_STEP):
        x = x_ref[img].astype(jnp.bfloat16)
        a = jnp.dot(w1m, x, preferred_element_type=jnp.float32)
        y1s.append(_lrelu(_combine_taps(a[:M1] + c1) + b1).astype(jnp.bfloat16))
        idents.append(a[M1:MBIG] + bp)

    y1cat = jnp.concatenate(y1s, axis=1)
    bcat = jnp.dot(w2m, y1cat, preferred_element_type=jnp.float32)

    for img in range(IMGS_PER_STEP):
        b = bcat[:, img * HW:(img + 1) * HW] + c2
        y2 = _lrelu(_combine_taps(b) + b2)
        o_ref[img] = ((y2 + idents[img]) * INV_SQRT2).reshape(PLANES, H, W)


def _tap_major(w):
    cout, cin = w.shape[0], w.shape[1]
    return w.transpose(2, 3, 0, 1).reshape(TAPS * cout, cin)


def kernel(w1, b1, w2, b2, wproj, bproj, x):
    B, Cin = x.shape[0], x.shape[1]
    x3 = x.astype(jnp.float32).reshape(B, Cin, HW)

    wcat = jnp.concatenate(
        [w1.astype(jnp.float32), w2.astype(jnp.float32)], axis=1)
    wt = jnp.concatenate(
        [_tap_major(wcat),
         jnp.pad(wproj.astype(jnp.float32).reshape(PLANES, Cin),
                 ((0, 0), (0, 36)))], axis=0)

    bias = jnp.stack([b1, b2, bproj], axis=1).astype(jnp.float32)

    span = jnp.arange(H, dtype=jnp.float32) / (H - 1) * 2.0 - 1.0
    xx = jnp.broadcast_to(span[:, None], (H, W)).reshape(1, HW)
    yy = jnp.broadcast_to(span[None, :], (H, W)).reshape(1, HW)
    coords = jnp.concatenate([xx, yy], axis=0).astype(jnp.bfloat16)

    out = pl.pallas_call(
        _block_kernel,
        grid=(B // IMGS_PER_STEP,),
        out_shape=jax.ShapeDtypeStruct((B, PLANES, H, W), jnp.float32),
        in_specs=[
            pl.BlockSpec((IMGS_PER_STEP, Cin, HW), lambda i: (i, 0, 0)),
            pl.BlockSpec(wt.shape, lambda i: (0, 0)),
            pl.BlockSpec(bias.shape, lambda i: (0, 0)),
            pl.BlockSpec(coords.shape, lambda i: (0, 0)),
        ],
        out_specs=pl.BlockSpec((IMGS_PER_STEP, PLANES, H, W),
                               lambda i: (i, 0, 0, 0)),
        compiler_params=pltpu.CompilerParams(
            dimension_semantics=("parallel",)),
    )(x3, wt, bias, coords)

    return out
